# baseline (speedup 1.0000x reference)
import functools

import jax
import jax.numpy as jnp
from jax import lax
from jax.experimental import pallas as pl
from jax.experimental.pallas import tpu as pltpu
from jax.experimental.pallas import tpu_sc as plsc

NC = 2
NS = 16
NW = NC * NS
CHUNK = 128
ALPHA = 0.1
KPROP = 10
ROW_BLK = 512


def _mlp_body(x_ref, w1_ref, b1_ref, w2_ref, b2_ref, o_ref):
    h = jnp.maximum(
        jnp.dot(x_ref[...], w1_ref[...], preferred_element_type=jnp.float32)
        + b1_ref[...], 0.0)
    o_ref[...] = (
        jnp.dot(h, w2_ref[...], preferred_element_type=jnp.float32)
        + b2_ref[...])


def _mlp(x_pad, W1, b1, W2, b2):
    n_pad, f_in = x_pad.shape
    f_hid = W1.shape[1]
    f_out = W2.shape[1]
    grid = (n_pad // ROW_BLK,)
    return pl.pallas_call(
        _mlp_body,
        grid=grid,
        in_specs=[
            pl.BlockSpec((ROW_BLK, f_in), lambda i: (i, 0)),
            pl.BlockSpec((f_in, f_hid), lambda i: (0, 0)),
            pl.BlockSpec((1, f_hid), lambda i: (0, 0)),
            pl.BlockSpec((f_hid, f_out), lambda i: (0, 0)),
            pl.BlockSpec((1, f_out), lambda i: (0, 0)),
        ],
        out_specs=pl.BlockSpec((ROW_BLK, f_out), lambda i: (i, 0)),
        out_shape=jax.ShapeDtypeStruct((n_pad, f_out), jnp.float32),
    )(x_pad, W1, b1.reshape(1, -1), W2, b2.reshape(1, -1))


def _prologue_body(pd0_ref, pd1_ref, h0_ref, dinv_ref, g_ref):
    deg = pd0_ref[...] + pd1_ref[...] + 1.0
    dinv16 = lax.rsqrt(deg)
    dinv = jnp.concatenate([dinv16] * (h0_ref.shape[1] // pd0_ref.shape[1]),
                           axis=1)
    dinv_ref[...] = dinv
    g_ref[...] = dinv * h0_ref[...]


def _prologue(pd0, pd1, h0):
    n_pad, f = h0.shape
    fd = pd0.shape[1]
    grid = (n_pad // ROW_BLK,)
    blk = pl.BlockSpec((ROW_BLK, f), lambda i: (i, 0))
    blkd = pl.BlockSpec((ROW_BLK, fd), lambda i: (i, 0))
    return pl.pallas_call(
        _prologue_body,
        grid=grid,
        in_specs=[blkd, blkd, blk],
        out_specs=[blk, blk],
        out_shape=[jax.ShapeDtypeStruct((n_pad, f), jnp.float32)] * 2,
    )(pd0, pd1, h0)


def _combine_body(p0_ref, p1_ref, g_ref, h0_ref, dinv_ref, h_ref, go_ref):
    s = p0_ref[...] + p1_ref[...] + g_ref[...]
    h = (1.0 - ALPHA) * (dinv_ref[...] * s) + ALPHA * h0_ref[...]
    h_ref[...] = h
    go_ref[...] = dinv_ref[...] * h


def _combine(p0, p1, g, h0, dinv):
    n_pad, f = h0.shape
    grid = (n_pad // ROW_BLK,)
    blk = pl.BlockSpec((ROW_BLK, f), lambda i: (i, 0))
    return pl.pallas_call(
        _combine_body,
        grid=grid,
        in_specs=[blk] * 5,
        out_specs=[blk, blk],
        out_shape=[jax.ShapeDtypeStruct((n_pad, f), jnp.float32)] * 2,
    )(p0, p1, g, h0, dinv)


NBUF = 4
GCH = 256


def _make_prop(n_pad, f, nch):
    hacc = n_pad + NS
    apt = hacc // NS
    opt = n_pad // NS
    nchg = nch // 2
    nchb = nchg // NBUF
    mesh = plsc.VectorSubcoreMesh(core_axis_name="c", subcore_axis_name="s")

    @functools.partial(
        pl.kernel,
        out_type=jax.ShapeDtypeStruct((NC, n_pad, f), jnp.float32),
        mesh=mesh,
        compiler_params=pltpu.CompilerParams(use_tc_tiling_on_sc=False),
        scratch_types=[
            pltpu.VMEM((nchg, GCH), jnp.int32),
            pltpu.VMEM((nch, CHUNK), jnp.int32),
            [pltpu.VMEM((GCH, f), jnp.float32)] * NBUF,
            pltpu.VMEM_SHARED((hacc, f), jnp.float32),
            [pltpu.SemaphoreType.DMA] * NBUF,
            [pltpu.SemaphoreType.DMA] * NBUF,
        ],
    )
    def prop(g_hbm, src_hbm, dst_hbm, zeros_hbm, out_hbm,
             src_v, dst_v, rows, acc_sh, gsem, ssem):
        c = lax.axis_index("c")
        s = lax.axis_index("s")
        wid = s * NC + c
        pltpu.sync_copy(src_hbm.at[wid], src_v)
        pltpu.sync_copy(dst_hbm.at[wid], dst_v)
        pltpu.sync_copy(zeros_hbm.at[pl.ds(s * apt, apt)],
                        acc_sh.at[pl.ds(s * apt, apt)])
        plsc.subcore_barrier()

        def body(t, carry):
            jb = t * NBUF
            for b in range(NBUF):
                @pl.when(t > 0)
                def _(b=b):
                    pltpu.make_async_copy(
                        rows[b].at[pl.ds(0, CHUNK)],
                        acc_sh.at[dst_v.at[2 * (jb + b)]], ssem[b]).wait()
                    pltpu.make_async_copy(
                        rows[b].at[pl.ds(CHUNK, CHUNK)],
                        acc_sh.at[dst_v.at[2 * (jb + b) + 1]], ssem[b]).wait()
                pltpu.async_copy(g_hbm.at[src_v.at[jb + b]], rows[b], gsem[b])
            for b in range(NBUF):
                pltpu.make_async_copy(
                    g_hbm.at[src_v.at[jb + b]], rows[b], gsem[b]).wait()
                pltpu.async_copy(
                    rows[b].at[pl.ds(0, CHUNK)],
                    acc_sh.at[dst_v.at[2 * (jb + b)]], ssem[b], add=True)
                pltpu.async_copy(
                    rows[b].at[pl.ds(CHUNK, CHUNK)],
                    acc_sh.at[dst_v.at[2 * (jb + b) + 1]], ssem[b], add=True)
            return carry

        lax.fori_loop(0, nchb, body, 0)
        for b in range(NBUF):
            jg = nchg - NBUF + b
            pltpu.make_async_copy(
                rows[b].at[pl.ds(0, CHUNK)],
                acc_sh.at[dst_v.at[2 * jg]], ssem[b]).wait()
            pltpu.make_async_copy(
                rows[b].at[pl.ds(CHUNK, CHUNK)],
                acc_sh.at[dst_v.at[2 * jg + 1]], ssem[b]).wait()
        plsc.subcore_barrier()
        pltpu.sync_copy(acc_sh.at[pl.ds(s * opt, opt)],
                        out_hbm.at[c, pl.ds(s * opt, opt)])

    return prop


def kernel(x, edge_index, W1, b1, W2, b2):
    n, f_in = x.shape
    e = edge_index.shape[1]
    f = W2.shape[1]

    n_pad = ((n + ROW_BLK - 1) // ROW_BLK) * ROW_BLK
    nch = (e + NW * CHUNK - 1) // (NW * CHUNK)
    nch = ((nch + 2 * NBUF - 1) // (2 * NBUF)) * (2 * NBUF)
    ew = nch * CHUNK
    e_pad = NW * ew

    x_pad = jnp.zeros((n_pad, f_in), x.dtype).at[:n].set(x)
    pad_e = e_pad - e
    wof = n_pad + (jnp.arange(pad_e, dtype=jnp.int32) % NS)
    src_p = jnp.concatenate(
        [edge_index[0].astype(jnp.int32), jnp.full((pad_e,), n, jnp.int32)])
    dst_flat = jnp.concatenate([edge_index[1].astype(jnp.int32), wof])
    src_p = src_p.reshape(NW, nch // 2, 2 * CHUNK)
    dst_p = dst_flat.reshape(NW, nch, CHUNK)
    zeros = jnp.zeros((n_pad, f), jnp.float32)
    ones16 = jnp.ones((n_pad, 16), jnp.float32)
    zeros16 = jnp.zeros((n_pad, 16), jnp.float32)

    prop = _make_prop(n_pad, f, nch)
    prop16 = _make_prop(n_pad, 16, nch)

    h0 = _mlp(x_pad, W1, b1, W2, b2)
    pd = prop16(ones16, src_p, dst_p, zeros16)
    dinv, g = _prologue(pd[0], pd[1], h0)
    h = h0
    for _ in range(KPROP):
        p = prop(g, src_p, dst_p, zeros)
        h, g = _combine(p[0], p[1], g, h0, dinv)
    return h[:n]


# speedup vs baseline: 1.0057x; 1.0057x over previous
import functools

import jax
import jax.numpy as jnp
from jax import lax
from jax.experimental import pallas as pl
from jax.experimental.pallas import tpu as pltpu
from jax.experimental.pallas import tpu_sc as plsc

NC = 2
NS = 16
NW = NC * NS
CHUNK = 128
ALPHA = 0.1
KPROP = 10
ROW_BLK = 512


def _mlp_body(x_ref, w1_ref, b1_ref, w2_ref, b2_ref, o_ref):
    h = jnp.maximum(
        jnp.dot(x_ref[...], w1_ref[...], preferred_element_type=jnp.float32)
        + b1_ref[...], 0.0)
    o_ref[...] = (
        jnp.dot(h, w2_ref[...], preferred_element_type=jnp.float32)
        + b2_ref[...])


def _mlp(x_pad, W1, b1, W2, b2):
    n_pad, f_in = x_pad.shape
    f_hid = W1.shape[1]
    f_out = W2.shape[1]
    grid = (n_pad // ROW_BLK,)
    return pl.pallas_call(
        _mlp_body,
        grid=grid,
        in_specs=[
            pl.BlockSpec((ROW_BLK, f_in), lambda i: (i, 0)),
            pl.BlockSpec((f_in, f_hid), lambda i: (0, 0)),
            pl.BlockSpec((1, f_hid), lambda i: (0, 0)),
            pl.BlockSpec((f_hid, f_out), lambda i: (0, 0)),
            pl.BlockSpec((1, f_out), lambda i: (0, 0)),
        ],
        out_specs=pl.BlockSpec((ROW_BLK, f_out), lambda i: (i, 0)),
        out_shape=jax.ShapeDtypeStruct((n_pad, f_out), jnp.float32),
    )(x_pad, W1, b1.reshape(1, -1), W2, b2.reshape(1, -1))


def _prologue_body(pd0_ref, pd1_ref, h0_ref, dinv_ref, g_ref):
    deg = pd0_ref[...] + pd1_ref[...] + 1.0
    dinv16 = lax.rsqrt(deg)
    dinv = jnp.concatenate([dinv16] * (h0_ref.shape[1] // pd0_ref.shape[1]),
                           axis=1)
    dinv_ref[...] = dinv
    g_ref[...] = dinv * h0_ref[...]


def _prologue(pd0, pd1, h0):
    n_pad, f = h0.shape
    fd = pd0.shape[1]
    grid = (n_pad // ROW_BLK,)
    blk = pl.BlockSpec((ROW_BLK, f), lambda i: (i, 0))
    blkd = pl.BlockSpec((ROW_BLK, fd), lambda i: (i, 0))
    return pl.pallas_call(
        _prologue_body,
        grid=grid,
        in_specs=[blkd, blkd, blk],
        out_specs=[blk, blk],
        out_shape=[jax.ShapeDtypeStruct((n_pad, f), jnp.float32)] * 2,
    )(pd0, pd1, h0)


def _combine_body(p0_ref, p1_ref, g_ref, h0_ref, dinv_ref, h_ref, go_ref):
    s = p0_ref[...] + p1_ref[...] + g_ref[...]
    h = (1.0 - ALPHA) * (dinv_ref[...] * s) + ALPHA * h0_ref[...]
    h_ref[...] = h
    go_ref[...] = dinv_ref[...] * h


def _combine(p0, p1, g, h0, dinv):
    n_pad, f = h0.shape
    grid = (n_pad // ROW_BLK,)
    blk = pl.BlockSpec((ROW_BLK, f), lambda i: (i, 0))
    return pl.pallas_call(
        _combine_body,
        grid=grid,
        in_specs=[blk] * 5,
        out_specs=[blk, blk],
        out_shape=[jax.ShapeDtypeStruct((n_pad, f), jnp.float32)] * 2,
    )(p0, p1, g, h0, dinv)


NBUF = 8


def _make_prop(n_pad, f, nch):
    hacc = n_pad + NS
    apt = hacc // NS
    opt = n_pad // NS
    nchb = nch // NBUF
    mesh = plsc.VectorSubcoreMesh(core_axis_name="c", subcore_axis_name="s")

    @functools.partial(
        pl.kernel,
        out_type=jax.ShapeDtypeStruct((NC, n_pad, f), jnp.float32),
        mesh=mesh,
        compiler_params=pltpu.CompilerParams(use_tc_tiling_on_sc=False),
        scratch_types=[
            pltpu.VMEM((nch, CHUNK), jnp.int32),
            pltpu.VMEM((nch, CHUNK), jnp.int32),
            [pltpu.VMEM((CHUNK, f), jnp.float32)] * NBUF,
            pltpu.VMEM_SHARED((hacc, f), jnp.float32),
            [pltpu.SemaphoreType.DMA] * NBUF,
            [pltpu.SemaphoreType.DMA] * NBUF,
        ],
    )
    def prop(g_hbm, src_hbm, dst_hbm, zeros_hbm, out_hbm,
             src_v, dst_v, rows, acc_sh, gsem, ssem):
        c = lax.axis_index("c")
        s = lax.axis_index("s")
        wid = s * NC + c
        pltpu.sync_copy(src_hbm.at[wid], src_v)
        pltpu.sync_copy(dst_hbm.at[wid], dst_v)
        pltpu.sync_copy(zeros_hbm.at[pl.ds(s * apt, apt)],
                        acc_sh.at[pl.ds(s * apt, apt)])
        plsc.subcore_barrier()

        def body(t, carry):
            jb = t * NBUF
            for b in range(NBUF):
                @pl.when(t > 0)
                def _(b=b):
                    pltpu.make_async_copy(
                        rows[b], acc_sh.at[dst_v.at[jb + b]], ssem[b]).wait()
                pltpu.async_copy(g_hbm.at[src_v.at[jb + b]], rows[b], gsem[b])
            for b in range(NBUF):
                pltpu.make_async_copy(
                    g_hbm.at[src_v.at[jb + b]], rows[b], gsem[b]).wait()
                pltpu.async_copy(
                    rows[b], acc_sh.at[dst_v.at[jb + b]], ssem[b], add=True)
            return carry

        lax.fori_loop(0, nchb, body, 0)
        for b in range(NBUF):
            pltpu.make_async_copy(
                rows[b], acc_sh.at[dst_v.at[nch - NBUF + b]], ssem[b]).wait()
        plsc.subcore_barrier()
        pltpu.sync_copy(acc_sh.at[pl.ds(s * opt, opt)],
                        out_hbm.at[c, pl.ds(s * opt, opt)])

    return prop


def kernel(x, edge_index, W1, b1, W2, b2):
    n, f_in = x.shape
    e = edge_index.shape[1]
    f = W2.shape[1]

    n_pad = ((n + ROW_BLK - 1) // ROW_BLK) * ROW_BLK
    nch = (e + NW * CHUNK - 1) // (NW * CHUNK)
    nch = ((nch + NBUF - 1) // NBUF) * NBUF
    ew = nch * CHUNK
    e_pad = NW * ew

    x_pad = jnp.zeros((n_pad, f_in), x.dtype).at[:n].set(x)
    pad_e = e_pad - e
    wof = n_pad + (jnp.arange(pad_e, dtype=jnp.int32) % NS)
    src_p = jnp.concatenate(
        [edge_index[0].astype(jnp.int32), jnp.full((pad_e,), n, jnp.int32)])
    dst_flat = jnp.concatenate([edge_index[1].astype(jnp.int32), wof])
    src_p = src_p.reshape(NW, nch, CHUNK)
    dst_p = dst_flat.reshape(NW, nch, CHUNK)
    zeros = jnp.zeros((n_pad, f), jnp.float32)
    ones16 = jnp.ones((n_pad, 16), jnp.float32)
    zeros16 = jnp.zeros((n_pad, 16), jnp.float32)

    prop = _make_prop(n_pad, f, nch)
    prop16 = _make_prop(n_pad, 16, nch)

    h0 = _mlp(x_pad, W1, b1, W2, b2)
    pd = prop16(ones16, src_p, dst_p, zeros16)
    dinv, g = _prologue(pd[0], pd[1], h0)
    h = h0
    for _ in range(KPROP):
        p = prop(g, src_p, dst_p, zeros)
        h, g = _combine(p[0], p[1], g, h0, dinv)
    return h[:n]
